# Initial kernel scaffold; baseline (speedup 1.0000x reference)
import jax
import jax.numpy as jnp
from jax.experimental import pallas as pl


def kernel(x, weight):
    raise NotImplementedError("write your pallas kernel here")


# speedup vs baseline: 4.0862x; 4.0862x over previous
import functools

import jax
import jax.numpy as jnp
from jax import lax
from jax.experimental import pallas as pl
from jax.experimental.pallas import tpu as pltpu
from jax.experimental.pallas import tpu_sc as plsc

_VOCAB = 1000
_DIM = 64
_TOTAL = 16384 * 200
_NC, _NS = 2, 16
_NW = _NC * _NS
_PER_W = _TOTAL // _NW
_IDXW = 128
_KSUB = 8
_CHUNK = _KSUB * _IDXW
_NCH = _PER_W // _CHUNK
_ROWS128 = _TOTAL // _IDXW


def _emb_body(x_hbm, w_hbm, out_hbm, idx_v, rows_v, sem):
    wid = lax.axis_index("s") * _NC + lax.axis_index("c")
    base128 = wid * (_PER_W // _IDXW)

    def chunk(ci, carry):
        r0 = base128 + ci * _KSUB
        pltpu.sync_copy(x_hbm.at[pl.ds(r0, _KSUB)], idx_v)
        cps = [
            pltpu.async_copy(w_hbm.at[idx_v.at[j]],
                             rows_v.at[pl.ds(j * _IDXW, _IDXW)], sem)
            for j in range(_KSUB)
        ]
        for cp in cps:
            cp.wait()
        pltpu.sync_copy(rows_v, out_hbm.at[pl.ds(r0 * _IDXW, _CHUNK)])
        return carry

    lax.fori_loop(0, _NCH, chunk, 0)


_emb = functools.partial(
    pl.kernel,
    mesh=plsc.VectorSubcoreMesh(core_axis_name="c", subcore_axis_name="s"),
    compiler_params=pltpu.CompilerParams(use_tc_tiling_on_sc=False),
    out_type=jax.ShapeDtypeStruct((_TOTAL, _DIM), jnp.float32),
    scratch_types=[
        pltpu.VMEM((_KSUB, _IDXW), jnp.int32),
        pltpu.VMEM((_CHUNK, _DIM), jnp.float32),
        pltpu.SemaphoreType.DMA,
    ],
)(_emb_body)


def kernel(x, weight):
    xf = x.reshape(_ROWS128, _IDXW)
    out = _emb(xf, weight)
    return out.reshape(x.shape[0], x.shape[1], _DIM)


# speedup vs baseline: 5.8121x; 1.4224x over previous
import functools

import jax
import jax.numpy as jnp
from jax import lax
from jax.experimental import pallas as pl
from jax.experimental.pallas import tpu as pltpu
from jax.experimental.pallas import tpu_sc as plsc

_VOCAB = 1000
_DIM = 64
_TOTAL = 16384 * 200
_NC, _NS = 2, 16
_NW = _NC * _NS
_PER_W = _TOTAL // _NW
_IDXW = 128
_KSUB = 4
_CHUNK = _KSUB * _IDXW
_NCH = _PER_W // _CHUNK
_ROWS128 = _TOTAL // _IDXW


def _emb_body(x_hbm, w_hbm, out_hbm, table_sh,
              idx0, idx1, rows0, rows1,
              sem_g, sem_i0, sem_i1, sem_s0, sem_s1):
    cid = lax.axis_index("c")
    sid = lax.axis_index("s")
    wid = sid * _NC + cid
    base = wid * (_PER_W // _IDXW)

    @pl.when(sid == 0)
    def _stage():
        pltpu.sync_copy(w_hbm, table_sh)
    plsc.subcore_barrier()

    idx_b = (idx0, idx1)
    rows_b = (rows0, rows1)
    sem_i = (sem_i0, sem_i1)
    sem_s = (sem_s0, sem_s1)

    def idx_slice(ci):
        return x_hbm.at[pl.ds(base + ci * _KSUB, _KSUB)]

    def out_slice(ci):
        return out_hbm.at[pl.ds((base + ci * _KSUB) * _IDXW, _CHUNK)]

    def do_gathers(b):
        cps = [pltpu.async_copy(table_sh.at[idx_b[b].at[j]],
                                rows_b[b].at[pl.ds(j * _IDXW, _IDXW)], sem_g)
               for j in range(_KSUB)]
        for cp in cps:
            cp.wait()

    h0 = pltpu.async_copy(idx_slice(0), idx0, sem_i0)
    h1 = pltpu.async_copy(idx_slice(1), idx1, sem_i1)
    for b, h in ((0, h0), (1, h1)):
        h.wait()
        do_gathers(b)
        pltpu.async_copy(rows_b[b], out_slice(b), sem_s[b])
        pltpu.async_copy(idx_slice(b + 2), idx_b[b], sem_i[b])

    def steady(k, carry):
        ci2 = 2 + 2 * k
        for b in range(2):
            ci = ci2 + b
            pltpu.make_async_copy(idx_slice(ci), idx_b[b], sem_i[b]).wait()
            pltpu.make_async_copy(rows_b[b], out_slice(ci), sem_s[b]).wait()
            do_gathers(b)
            pltpu.async_copy(rows_b[b], out_slice(ci), sem_s[b])
            pltpu.async_copy(idx_slice(ci + 2), idx_b[b], sem_i[b])
        return carry

    lax.fori_loop(0, (_NCH - 4) // 2, steady, 0)

    for b in range(2):
        ci = _NCH - 2 + b
        pltpu.make_async_copy(idx_slice(ci), idx_b[b], sem_i[b]).wait()
        pltpu.make_async_copy(rows_b[b], out_slice(ci), sem_s[b]).wait()
        do_gathers(b)
        pltpu.async_copy(rows_b[b], out_slice(ci), sem_s[b])
    for b in range(2):
        pltpu.make_async_copy(rows_b[b], out_slice(_NCH - 2 + b), sem_s[b]).wait()


_emb = functools.partial(
    pl.kernel,
    mesh=plsc.VectorSubcoreMesh(core_axis_name="c", subcore_axis_name="s"),
    compiler_params=pltpu.CompilerParams(use_tc_tiling_on_sc=False),
    out_type=jax.ShapeDtypeStruct((_TOTAL, _DIM), jnp.float32),
    scratch_types=[
        pltpu.MemorySpace.VMEM_SHARED((_VOCAB, _DIM), jnp.float32),
        pltpu.VMEM((_KSUB, _IDXW), jnp.int32),
        pltpu.VMEM((_KSUB, _IDXW), jnp.int32),
        pltpu.VMEM((_CHUNK, _DIM), jnp.float32),
        pltpu.VMEM((_CHUNK, _DIM), jnp.float32),
        pltpu.SemaphoreType.DMA,
        pltpu.SemaphoreType.DMA,
        pltpu.SemaphoreType.DMA,
        pltpu.SemaphoreType.DMA,
        pltpu.SemaphoreType.DMA,
    ],
)(_emb_body)


def kernel(x, weight):
    xf = x.reshape(_ROWS128, _IDXW)
    out = _emb(xf, weight)
    return out.reshape(x.shape[0], x.shape[1], _DIM)
